# baseline (speedup 1.0000x reference)
import functools

import jax
import jax.numpy as jnp
from jax import lax
from jax.experimental import pallas as pl
from jax.experimental.pallas import tpu as pltpu
from jax.experimental.pallas import tpu_sc as plsc

EMB = 32
G = 128
NC, NS = 2, 16
NW = NC * NS
H = 50
B = 16384
NBLK = H * (B // G)


def _sc_gather_add_t():
    blocks_per_w = NBLK // NW

    mesh = plsc.VectorSubcoreMesh(core_axis_name="c", subcore_axis_name="s")

    @functools.partial(
        pl.kernel,
        mesh=mesh,
        out_type=jax.ShapeDtypeStruct((H, 4, B // G, 8, G), jnp.float32),
        compiler_params=pltpu.CompilerParams(
            use_tc_tiling_on_sc=False, needs_layout_passes=False
        ),
        scratch_types=[
            pltpu.VMEM((2, G), jnp.int32),
            pltpu.VMEM((2, G), jnp.int32),
            pltpu.VMEM((2, G, EMB), jnp.float32),
            pltpu.VMEM((2, G, EMB), jnp.float32),
            pltpu.VMEM((2, 4, 8, G), jnp.float32),
            pltpu.SemaphoreType.DMA,
            pltpu.SemaphoreType.DMA,
            pltpu.SemaphoreType.DMA,
            pltpu.SemaphoreType.DMA,
            pltpu.SemaphoreType.DMA,
            pltpu.SemaphoreType.DMA,
        ],
    )
    def k(i1_hbm, i2_hbm, w1_hbm, w2_hbm, o_hbm,
          i1v, i2v, r1v, r2v, ov, sg0, sg1, si0, si1, ss0, ss1):
        sem_g = (sg0, sg1)
        sem_i = (si0, si1)
        sem_s = (ss0, ss1)
        wid = lax.axis_index("s") * NC + lax.axis_index("c")
        gbase = wid * blocks_per_w

        iota = lax.iota(jnp.int32, 16)
        jrows = [iota + (16 * j) for j in range(G // 16)]

        def fire_idx(g, s):
            pltpu.async_copy(i1_hbm.at[pl.ds(g * G, G)], i1v.at[s], sem_i[s])
            pltpu.async_copy(i2_hbm.at[pl.ds(g * G, G)], i2v.at[s], sem_i[s])

        def wait_idx(s):
            pltpu.make_async_copy(i1_hbm.at[pl.ds(0, G)], i1v.at[s], sem_i[s]).wait()
            pltpu.make_async_copy(i2_hbm.at[pl.ds(0, G)], i2v.at[s], sem_i[s]).wait()

        def fire_gathers(s):
            pltpu.async_copy(w1_hbm.at[i1v.at[s]], r1v.at[s], sem_g[s])
            pltpu.async_copy(w2_hbm.at[i2v.at[s]], r2v.at[s], sem_g[s])

        def drain_gathers(s):
            pltpu.make_async_copy(w1_hbm.at[i1v.at[s]], r1v.at[s], sem_g[s]).wait()
            pltpu.make_async_copy(w2_hbm.at[i2v.at[s]], r2v.at[s], sem_g[s]).wait()

        def fire_store(g, s):
            h = g >> 7
            b128 = g & 127
            pltpu.async_copy(ov.at[s], o_hbm.at[h, :, b128], sem_s[s])

        def wait_store(s):
            pltpu.make_async_copy(ov.at[s], o_hbm.at[0, :, 0], sem_s[s]).wait()

        def compute(s):
            for c8 in range(4):
                for sub in range(8):
                    c = 8 * c8 + sub
                    ccol = jnp.full((16,), c, jnp.int32)
                    for j in range(G // 16):
                        a = plsc.load_gather(r1v.at[s], [jrows[j], ccol])
                        b = plsc.load_gather(r2v.at[s], [jrows[j], ccol])
                        ov[s, c8, sub, pl.ds(16 * j, 16)] = a + b

        pltpu.sync_copy(i1_hbm.at[pl.ds(gbase * G, G)], i1v.at[0])
        pltpu.sync_copy(i2_hbm.at[pl.ds(gbase * G, G)], i2v.at[0])
        fire_gathers(0)
        fire_idx(gbase + 1, 1)

        @pl.loop(0, blocks_per_w // 2)
        def _(p):
            for sl in range(2):
                gl = 2 * p + sl
                g = gbase + gl
                o = 1 - sl

                @pl.when(gl + 1 < blocks_per_w)
                def _():
                    wait_idx(o)
                    fire_gathers(o)

                drain_gathers(sl)

                @pl.when(gl >= 2)
                def _():
                    wait_store(sl)

                compute(sl)
                fire_store(g, sl)

                @pl.when(gl + 2 < blocks_per_w)
                def _():
                    fire_idx(g + 2, sl)

        wait_store(0)
        wait_store(1)

    return k


def kernel(input, another_input, W1, W2):
    i1t = jnp.transpose(input.astype(jnp.int32)).reshape(H * B)
    i2t = jnp.transpose(another_input.astype(jnp.int32)).reshape(H * B)
    out5 = _sc_gather_add_t()(i1t, i2t, W1, W2)
    out = out5.transpose(0, 1, 3, 2, 4).reshape(H, EMB, B).transpose(2, 0, 1)
    return out


# speedup vs baseline: 1.1510x; 1.1510x over previous
import functools

import jax
import jax.numpy as jnp
from jax import lax
from jax.experimental import pallas as pl
from jax.experimental.pallas import tpu as pltpu
from jax.experimental.pallas import tpu_sc as plsc

EMB = 32
G = 128
NC, NS = 2, 16
NW = NC * NS
H = 50
B = 16384
NBLK = H * (B // G)


def _sc_gather_add_t():
    blocks_per_w = NBLK // NW

    mesh = plsc.VectorSubcoreMesh(core_axis_name="c", subcore_axis_name="s")

    @functools.partial(
        pl.kernel,
        mesh=mesh,
        out_type=jax.ShapeDtypeStruct((H, 4, B // G, 8, G), jnp.float32),
        compiler_params=pltpu.CompilerParams(
            use_tc_tiling_on_sc=False, needs_layout_passes=False
        ),
        scratch_types=[
            pltpu.VMEM((2, G), jnp.int32),
            pltpu.VMEM((2, G), jnp.int32),
            pltpu.VMEM((2, G, EMB), jnp.float32),
            pltpu.VMEM((2, G, EMB), jnp.float32),
            pltpu.VMEM((2, 4, 8, G), jnp.float32),
            pltpu.SemaphoreType.DMA,
            pltpu.SemaphoreType.DMA,
            pltpu.SemaphoreType.DMA,
            pltpu.SemaphoreType.DMA,
            pltpu.SemaphoreType.DMA,
            pltpu.SemaphoreType.DMA,
        ],
    )
    def k(i1_hbm, i2_hbm, w1_hbm, w2_hbm, o_hbm,
          i1v, i2v, r1v, r2v, ov, sg0, sg1, si0, si1, ss0, ss1):
        sem_g = (sg0, sg1)
        sem_i = (si0, si1)
        sem_s = (ss0, ss1)
        wid = lax.axis_index("s") * NC + lax.axis_index("c")
        gbase = wid * blocks_per_w

        iota = lax.iota(jnp.int32, 16)

        def fire_idx(g, s):
            pltpu.async_copy(i1_hbm.at[pl.ds(g * G, G)], i1v.at[s], sem_i[s])
            pltpu.async_copy(i2_hbm.at[pl.ds(g * G, G)], i2v.at[s], sem_i[s])

        def wait_idx(s):
            pltpu.make_async_copy(i1_hbm.at[pl.ds(0, G)], i1v.at[s], sem_i[s]).wait()
            pltpu.make_async_copy(i2_hbm.at[pl.ds(0, G)], i2v.at[s], sem_i[s]).wait()

        def fire_gathers(s):
            pltpu.async_copy(w1_hbm.at[i1v.at[s]], r1v.at[s], sem_g[s])
            pltpu.async_copy(w2_hbm.at[i2v.at[s]], r2v.at[s], sem_g[s])

        def drain_gathers(s):
            pltpu.make_async_copy(w1_hbm.at[i1v.at[s]], r1v.at[s], sem_g[s]).wait()
            pltpu.make_async_copy(w2_hbm.at[i2v.at[s]], r2v.at[s], sem_g[s]).wait()

        def fire_store(g, s):
            h = g >> 7
            b128 = g & 127
            pltpu.async_copy(ov.at[s], o_hbm.at[h, :, b128], sem_s[s])

        def wait_store(s):
            pltpu.make_async_copy(ov.at[s], o_hbm.at[0, :, 0], sem_s[s]).wait()

        def compute(s):
            @plsc.parallel_loop(0, (G // 16) * EMB, 1, unroll=8)
            def _(t):
                jrow = iota + ((t & 7) << 4)
                ccol = jnp.full((16,), 0, jnp.int32) + (t >> 3)
                a = plsc.load_gather(r1v.at[s], [jrow, ccol])
                b = plsc.load_gather(r2v.at[s], [jrow, ccol])
                ov[s, t >> 6, (t >> 3) & 7, pl.ds((t & 7) * 16, 16)] = a + b

        pltpu.sync_copy(i1_hbm.at[pl.ds(gbase * G, G)], i1v.at[0])
        pltpu.sync_copy(i2_hbm.at[pl.ds(gbase * G, G)], i2v.at[0])
        fire_gathers(0)
        fire_idx(gbase + 1, 1)

        @pl.loop(0, blocks_per_w // 2)
        def _(p):
            for sl in range(2):
                gl = 2 * p + sl
                g = gbase + gl
                o = 1 - sl

                @pl.when(gl + 1 < blocks_per_w)
                def _():
                    wait_idx(o)
                    fire_gathers(o)

                drain_gathers(sl)

                @pl.when(gl >= 2)
                def _():
                    wait_store(sl)

                compute(sl)
                fire_store(g, sl)

                @pl.when(gl + 2 < blocks_per_w)
                def _():
                    fire_idx(g + 2, sl)

        wait_store(0)
        wait_store(1)

    return k


def kernel(input, another_input, W1, W2):
    i1t = jnp.transpose(input.astype(jnp.int32)).reshape(H * B)
    i2t = jnp.transpose(another_input.astype(jnp.int32)).reshape(H * B)
    out5 = _sc_gather_add_t()(i1t, i2t, W1, W2)
    out = out5.transpose(0, 1, 3, 2, 4).reshape(H, EMB, B).transpose(2, 0, 1)
    return out


# speedup vs baseline: 1.1562x; 1.0045x over previous
import functools

import jax
import jax.numpy as jnp
from jax import lax
from jax.experimental import pallas as pl
from jax.experimental.pallas import tpu as pltpu
from jax.experimental.pallas import tpu_sc as plsc

EMB = 32
G = 128
NC, NS = 2, 16
NW = NC * NS
H = 50
B = 16384
NBLK = H * (B // G)


def _sc_gather_add_t():
    blocks_per_w = NBLK // NW

    mesh = plsc.VectorSubcoreMesh(core_axis_name="c", subcore_axis_name="s")

    @functools.partial(
        pl.kernel,
        mesh=mesh,
        out_type=jax.ShapeDtypeStruct((H, 4, B // G, 8, G), jnp.float32),
        compiler_params=pltpu.CompilerParams(
            use_tc_tiling_on_sc=False, needs_layout_passes=False
        ),
        scratch_types=[
            pltpu.VMEM((2, G), jnp.int32),
            pltpu.VMEM((2, G), jnp.int32),
            pltpu.VMEM((2, G, EMB), jnp.float32),
            pltpu.VMEM((2, G, EMB), jnp.float32),
            pltpu.VMEM((2, 4, 8, G), jnp.float32),
            pltpu.SemaphoreType.DMA,
            pltpu.SemaphoreType.DMA,
            pltpu.SemaphoreType.DMA,
            pltpu.SemaphoreType.DMA,
            pltpu.SemaphoreType.DMA,
            pltpu.SemaphoreType.DMA,
        ],
    )
    def k(i1_hbm, i2_hbm, w1_hbm, w2_hbm, o_hbm,
          i1v, i2v, r1v, r2v, ov, sg0, sg1, si0, si1, ss0, ss1):
        sem_g = (sg0, sg1)
        sem_i = (si0, si1)
        sem_s = (ss0, ss1)
        wid = lax.axis_index("s") * NC + lax.axis_index("c")
        gbase = wid * blocks_per_w

        iota = lax.iota(jnp.int32, 16)

        def fire_idx(g, s):
            pltpu.async_copy(i1_hbm.at[pl.ds(g * G, G)], i1v.at[s], sem_i[s])
            pltpu.async_copy(i2_hbm.at[pl.ds(g * G, G)], i2v.at[s], sem_i[s])

        def wait_idx(s):
            pltpu.make_async_copy(i1_hbm.at[pl.ds(0, G)], i1v.at[s], sem_i[s]).wait()
            pltpu.make_async_copy(i2_hbm.at[pl.ds(0, G)], i2v.at[s], sem_i[s]).wait()

        def fire_gathers(s):
            pltpu.async_copy(w1_hbm.at[i1v.at[s]], r1v.at[s], sem_g[s])
            pltpu.async_copy(w2_hbm.at[i2v.at[s]], r2v.at[s], sem_g[s])

        def drain_gathers(s):
            pltpu.make_async_copy(w1_hbm.at[i1v.at[s]], r1v.at[s], sem_g[s]).wait()
            pltpu.make_async_copy(w2_hbm.at[i2v.at[s]], r2v.at[s], sem_g[s]).wait()

        def fire_store(g, s):
            h = g >> 7
            b128 = g & 127
            pltpu.async_copy(ov.at[s], o_hbm.at[h, :, b128], sem_s[s])

        def wait_store(s):
            pltpu.make_async_copy(ov.at[s], o_hbm.at[0, :, 0], sem_s[s]).wait()

        def compute(s):
            @plsc.parallel_loop(0, (G // 16) * EMB, 1, unroll=16)
            def _(t):
                jrow = iota + ((t & 7) << 4)
                ccol = jnp.full((16,), 0, jnp.int32) + (t >> 3)
                a = plsc.load_gather(r1v.at[s], [jrow, ccol])
                b = plsc.load_gather(r2v.at[s], [jrow, ccol])
                ov[s, t >> 6, (t >> 3) & 7, pl.ds((t & 7) * 16, 16)] = a + b

        pltpu.sync_copy(i1_hbm.at[pl.ds(gbase * G, G)], i1v.at[0])
        pltpu.sync_copy(i2_hbm.at[pl.ds(gbase * G, G)], i2v.at[0])
        fire_gathers(0)
        fire_idx(gbase + 1, 1)

        @pl.loop(0, blocks_per_w // 2)
        def _(p):
            for sl in range(2):
                gl = 2 * p + sl
                g = gbase + gl
                o = 1 - sl

                @pl.when(gl + 1 < blocks_per_w)
                def _():
                    wait_idx(o)
                    fire_gathers(o)

                drain_gathers(sl)

                @pl.when(gl >= 2)
                def _():
                    wait_store(sl)

                compute(sl)
                fire_store(g, sl)

                @pl.when(gl + 2 < blocks_per_w)
                def _():
                    fire_idx(g + 2, sl)

        wait_store(0)
        wait_store(1)

    return k


def kernel(input, another_input, W1, W2):
    i1t = jnp.transpose(input.astype(jnp.int32)).reshape(H * B)
    i2t = jnp.transpose(another_input.astype(jnp.int32)).reshape(H * B)
    out5 = _sc_gather_add_t()(i1t, i2t, W1, W2)
    out = out5.transpose(0, 1, 3, 2, 4).reshape(H, EMB, B).transpose(2, 0, 1)
    return out
